# Initial kernel scaffold; baseline (speedup 1.0000x reference)
import jax
import jax.numpy as jnp
from jax.experimental import pallas as pl


def kernel(indices, w, b):
    raise NotImplementedError("write your pallas kernel here")


# speedup vs baseline: 1.4393x; 1.4393x over previous
import jax
import jax.numpy as jnp
from jax import lax
from jax.experimental import pallas as pl
from jax.experimental.pallas import tpu as pltpu
from jax.experimental.pallas import tpu_sc as plsc

B = 16384
F = 26
NW = 32
BPW = B // NW
IPW = BPW * F
L = 16


def _sc_body(idx_hbm, w_hbm, b_hbm, out_hbm, idx_v, vals_v, b_v, out_v, sem):
    wid = lax.axis_index("s") * 2 + lax.axis_index("c")
    pltpu.sync_copy(b_hbm, b_v)
    pltpu.sync_copy(idx_hbm.at[wid], idx_v)
    pltpu.async_copy(w_hbm.at[idx_v], vals_v, sem).wait()

    bias = b_v[...]

    @pl.loop(0, BPW, step=L)
    def _(c):
        acc = bias
        for f in range(F):
            acc = acc + vals_v[pl.ds(f * BPW + c, L)]
        out_v[pl.ds(c, L)] = 1.0 / (1.0 + jnp.exp(-acc))

    pltpu.sync_copy(out_v, out_hbm.at[pl.ds(wid * BPW, BPW)])


def kernel(indices, w, b):
    idx_r = (
        indices.astype(jnp.int32)
        .reshape(NW, BPW, F)
        .swapaxes(1, 2)
        .reshape(NW, IPW)
    )
    w_flat = w.reshape(-1)
    b16 = jnp.broadcast_to(b.astype(jnp.float32), (L,))

    mesh = plsc.VectorSubcoreMesh(core_axis_name="c", subcore_axis_name="s")
    sc_fn = pl.kernel(
        _sc_body,
        out_type=jax.ShapeDtypeStruct((B,), jnp.float32),
        mesh=mesh,
        scratch_types=[
            pltpu.VMEM((IPW,), jnp.int32),
            pltpu.VMEM((IPW,), jnp.float32),
            pltpu.VMEM((L,), jnp.float32),
            pltpu.VMEM((BPW,), jnp.float32),
            pltpu.SemaphoreType.DMA,
        ],
    )
    return sc_fn(idx_r, w_flat, b16)


# speedup vs baseline: 2.4909x; 1.7307x over previous
import dataclasses

import jax
import jax.numpy as jnp
from jax import lax
from jax.experimental import pallas as pl
from jax.experimental.pallas import tpu as pltpu
from jax.experimental.pallas import tpu_sc as plsc

B = 16384
F = 26
NW = 32
BPW = B // NW
IPW = BPW * F
L = 16
INPUT_ROWS = 1000000
WPAD = 1000448


def _sc_body(idx_hbm, w_hbm, b_hbm, out_hbm, idx_v, vals_v, b_v, out_v, sem):
    wid = lax.axis_index("s") * 2 + lax.axis_index("c")
    pltpu.sync_copy(b_hbm, b_v)
    pltpu.sync_copy(idx_hbm.at[wid], idx_v)
    pltpu.async_copy(w_hbm.at[idx_v], vals_v, sem).wait()

    bias = b_v[...]

    @pl.loop(0, BPW, step=L)
    def _(c):
        acc = bias
        for f in range(F):
            acc = acc + vals_v[pl.ds(f * BPW + c, L)]
        out_v[pl.ds(c, L)] = 1.0 / (1.0 + jnp.exp(-acc))

    pltpu.sync_copy(out_v, out_hbm.at[pl.ds(wid * BPW, BPW)])


def kernel(indices, w, b):
    idx_r = (
        indices.astype(jnp.int32)
        .reshape(NW, BPW, F)
        .swapaxes(1, 2)
        .reshape(NW, IPW)
    )
    b16 = jnp.broadcast_to(b.astype(jnp.float32), (L,))
    w_pad = jnp.concatenate([w, jnp.zeros((WPAD - INPUT_ROWS, 1), jnp.float32)])
    w_flat = w_pad.reshape(WPAD)

    cp = pltpu.CompilerParams()
    if "needs_layout_passes" in pltpu.CompilerParams.__dataclass_fields__:
        cp = dataclasses.replace(cp, needs_layout_passes=False)
    mesh = plsc.VectorSubcoreMesh(core_axis_name="c", subcore_axis_name="s")
    sc_fn = pl.kernel(
        _sc_body,
        out_type=jax.ShapeDtypeStruct((B,), jnp.float32),
        mesh=mesh,
        compiler_params=cp,
        scratch_types=[
            pltpu.VMEM((IPW,), jnp.int32),
            pltpu.VMEM((IPW,), jnp.float32),
            pltpu.VMEM((L,), jnp.float32),
            pltpu.VMEM((BPW,), jnp.float32),
            pltpu.SemaphoreType.DMA,
        ],
    )
    return sc_fn(idx_r, w_flat, b16)


# speedup vs baseline: 2.6249x; 1.0538x over previous
import dataclasses

import jax
import jax.numpy as jnp
from jax import lax
from jax.experimental import pallas as pl
from jax.experimental.pallas import tpu as pltpu
from jax.experimental.pallas import tpu_sc as plsc

B = 16384
F = 26
NW = 32
BPW = B // NW
IPW = BPW * F
L = 16
INPUT_ROWS = 1000000
WPAD = 1000448


def _sc_body(idx_hbm, w_hbm, b_hbm, out_hbm, idx_v, vals_v, b_v, out_v, sem, gsem):
    wid = lax.axis_index("s") * 2 + lax.axis_index("c")
    pltpu.sync_copy(b_hbm, b_v)
    row_copies = [
        pltpu.async_copy(
            idx_hbm.at[f, pl.ds(wid * BPW, BPW)], idx_v.at[pl.ds(f * BPW, BPW)], sem
        )
        for f in range(F)
    ]
    for c in row_copies:
        c.wait()
    pltpu.async_copy(w_hbm.at[idx_v], vals_v, gsem).wait()

    bias = b_v[...]

    @pl.loop(0, BPW, step=L)
    def _(c):
        acc = bias
        for f in range(F):
            acc = acc + vals_v[pl.ds(f * BPW + c, L)]
        out_v[pl.ds(c, L)] = 1.0 / (1.0 + jnp.exp(-acc))

    pltpu.sync_copy(out_v, out_hbm.at[pl.ds(wid * BPW, BPW)])


def kernel(indices, w, b):
    idx_t = indices.astype(jnp.int32).T
    b16 = jnp.broadcast_to(b.astype(jnp.float32), (L,))
    w_pad = jnp.concatenate([w, jnp.zeros((WPAD - INPUT_ROWS, 1), jnp.float32)])
    w_flat = w_pad.reshape(WPAD)

    cp = pltpu.CompilerParams()
    if "needs_layout_passes" in pltpu.CompilerParams.__dataclass_fields__:
        cp = dataclasses.replace(cp, needs_layout_passes=False)
    mesh = plsc.VectorSubcoreMesh(core_axis_name="c", subcore_axis_name="s")
    sc_fn = pl.kernel(
        _sc_body,
        out_type=jax.ShapeDtypeStruct((B,), jnp.float32),
        mesh=mesh,
        compiler_params=cp,
        scratch_types=[
            pltpu.VMEM((IPW,), jnp.int32),
            pltpu.VMEM((IPW,), jnp.float32),
            pltpu.VMEM((L,), jnp.float32),
            pltpu.VMEM((BPW,), jnp.float32),
            pltpu.SemaphoreType.DMA,
            pltpu.SemaphoreType.DMA,
        ],
    )
    return sc_fn(idx_t, w_flat, b16)
